# baseline (speedup 1.0000x reference)
import functools

import jax
import jax.numpy as jnp
from jax import lax
from jax.experimental import pallas as pl
from jax.experimental.pallas import tpu as pltpu
from jax.experimental.pallas import tpu_sc as plsc


def _sc_gather(table, idx):
    B = idx.shape[0]
    D = table.shape[1]
    info = plsc.get_sparse_core_info()
    nw = info.num_cores * info.num_subcores
    b_per_w = B // nw
    mesh = plsc.VectorSubcoreMesh(core_axis_name="c", subcore_axis_name="s")

    @functools.partial(
        pl.kernel,
        mesh=mesh,
        out_type=jax.ShapeDtypeStruct((B, D), jnp.float32),
        scratch_types=[
            pltpu.VMEM((b_per_w,), jnp.int32),
            pltpu.VMEM((b_per_w, D), jnp.float32),
            pltpu.SemaphoreType.DMA,
        ],
        compiler_params=pltpu.CompilerParams(use_tc_tiling_on_sc=False),
    )
    def gather_kernel(table_hbm, idx_hbm, out_hbm, idx_v, rows_v, sem):
        wid = lax.axis_index("s") * info.num_cores + lax.axis_index("c")
        base = wid * b_per_w
        pltpu.sync_copy(idx_hbm.at[pl.ds(base, b_per_w)], idx_v)
        pltpu.async_copy(table_hbm.at[idx_v], rows_v, sem).wait()
        pltpu.sync_copy(rows_v, out_hbm.at[pl.ds(base, b_per_w)])

    return gather_kernel(table, idx)


def _tc_project(h, lin_w, lin_b, bt=16, nbuf=6):
    B, E = h.shape
    V = lin_w.shape[0]
    nsteps = B // bt
    wt = lin_w.astype(jnp.bfloat16).T
    hb = h.astype(jnp.bfloat16)

    def mm(h_ref, w_ref, b_ref, o_ref, ring, sems):
        i = pl.program_id(0)
        slot = lax.rem(i, nbuf)

        for k in range(nbuf):
            @pl.when(jnp.logical_and(slot == k, i >= nbuf))
            def _wait_prev():
                pltpu.make_async_copy(
                    ring.at[k], o_ref.at[pl.ds(0, bt), :], sems.at[k]
                ).wait()

            @pl.when(slot == k)
            def _start():
                pltpu.make_async_copy(
                    ring.at[k], o_ref.at[pl.ds(i * bt, bt), :], sems.at[k]
                ).start()

        @pl.when(i == nsteps - 1)
        def _drain():
            for k in range(nbuf):
                pltpu.make_async_copy(
                    ring.at[k], o_ref.at[pl.ds(0, bt), :], sems.at[k]
                ).wait()

    return pl.pallas_call(
        mm,
        grid=(nsteps,),
        in_specs=[
            pl.BlockSpec((bt, E), lambda i: (i, 0)),
            pl.BlockSpec((E, V), lambda i: (0, 0),
                         pipeline_mode=pl.Buffered(buffer_count=1)),
            pl.BlockSpec((V,), lambda i: (0,),
                         pipeline_mode=pl.Buffered(buffer_count=1)),
        ],
        out_specs=pl.BlockSpec(memory_space=pltpu.MemorySpace.HBM),
        out_shape=jax.ShapeDtypeStruct((B, V), jnp.float32),
        scratch_shapes=[
            pltpu.VMEM((nbuf, bt, V), jnp.float32),
            pltpu.SemaphoreType.DMA((nbuf,)),
        ],
        compiler_params=pltpu.CompilerParams(
            vmem_limit_bytes=100 * 1024 * 1024,
        ),
    )(hb, wt, lin_b)


def kernel(x, emb_table, lin_w, lin_b):
    h = jnp.take(emb_table, x, axis=0)
    return _tc_project(h, lin_w, lin_b)


# speedup vs baseline: 2.3849x; 2.3849x over previous
import functools

import jax
import jax.numpy as jnp
from jax import lax
from jax.experimental import pallas as pl
from jax.experimental.pallas import tpu as pltpu
from jax.experimental.pallas import tpu_sc as plsc


def _sc_gather(table, idx):
    B = idx.shape[0]
    D = table.shape[1]
    info = plsc.get_sparse_core_info()
    nw = info.num_cores * info.num_subcores
    b_per_w = B // nw
    mesh = plsc.VectorSubcoreMesh(core_axis_name="c", subcore_axis_name="s")

    @functools.partial(
        pl.kernel,
        mesh=mesh,
        out_type=jax.ShapeDtypeStruct((B, D), jnp.float32),
        scratch_types=[
            pltpu.VMEM((b_per_w,), jnp.int32),
            pltpu.VMEM((b_per_w, D), jnp.float32),
            pltpu.SemaphoreType.DMA,
        ],
        compiler_params=pltpu.CompilerParams(use_tc_tiling_on_sc=False),
    )
    def gather_kernel(table_hbm, idx_hbm, out_hbm, idx_v, rows_v, sem):
        wid = lax.axis_index("s") * info.num_cores + lax.axis_index("c")
        base = wid * b_per_w
        pltpu.sync_copy(idx_hbm.at[pl.ds(base, b_per_w)], idx_v)
        pltpu.async_copy(table_hbm.at[idx_v], rows_v, sem).wait()
        pltpu.sync_copy(rows_v, out_hbm.at[pl.ds(base, b_per_w)])

    return gather_kernel(table, idx)


def _tc_project(h, lin_w, lin_b, vt=2048):
    B, E = h.shape
    V = lin_w.shape[0]

    def mm(w_ref, h_ref, b_ref, o_ref):
        acc = lax.dot_general(
            w_ref[...], h_ref[...],
            (((1,), (1,)), ((), ())),
            preferred_element_type=jnp.float32,
        )
        o_ref[...] = acc + b_ref[...][:, None]

    out_t = pl.pallas_call(
        mm,
        grid=(pl.cdiv(V, vt),),
        in_specs=[
            pl.BlockSpec((vt, E), lambda i: (i, 0)),
            pl.BlockSpec((B, E), lambda i: (0, 0)),
            pl.BlockSpec((vt,), lambda i: (i,)),
        ],
        out_specs=pl.BlockSpec((vt, B), lambda i: (i, 0)),
        out_shape=jax.ShapeDtypeStruct((V, B), jnp.float32),
    )(lin_w, h, lin_b)
    return out_t.T


def kernel(x, emb_table, lin_w, lin_b):
    h = jnp.take(emb_table, x, axis=0)
    return _tc_project(h, lin_w, lin_b)


# speedup vs baseline: 2.8273x; 1.1855x over previous
import functools

import jax
import jax.numpy as jnp
from jax import lax
from jax.experimental import pallas as pl
from jax.experimental.pallas import tpu as pltpu
from jax.experimental.pallas import tpu_sc as plsc


def _sc_gather(table, idx):
    B = idx.shape[0]
    D = table.shape[1]
    info = plsc.get_sparse_core_info()
    nw = info.num_cores * info.num_subcores
    b_per_w = B // nw
    mesh = plsc.VectorSubcoreMesh(core_axis_name="c", subcore_axis_name="s")

    @functools.partial(
        pl.kernel,
        mesh=mesh,
        out_type=jax.ShapeDtypeStruct((B, D), jnp.float32),
        scratch_types=[
            pltpu.VMEM((b_per_w,), jnp.int32),
            pltpu.VMEM((b_per_w, D), jnp.float32),
            pltpu.SemaphoreType.DMA,
        ],
        compiler_params=pltpu.CompilerParams(use_tc_tiling_on_sc=False),
    )
    def gather_kernel(table_hbm, idx_hbm, out_hbm, idx_v, rows_v, sem):
        wid = lax.axis_index("s") * info.num_cores + lax.axis_index("c")
        base = wid * b_per_w
        pltpu.sync_copy(idx_hbm.at[pl.ds(base, b_per_w)], idx_v)
        pltpu.async_copy(table_hbm.at[idx_v], rows_v, sem).wait()
        pltpu.sync_copy(rows_v, out_hbm.at[pl.ds(base, b_per_w)])

    return gather_kernel(table, idx)


def _tc_project(h, lin_w, lin_b, vt=2048):
    B, E = h.shape
    V = lin_w.shape[0]

    def mm(w_ref, h_ref, b_ref, o_ref):
        acc = lax.dot_general(
            w_ref[...], h_ref[...],
            (((0,), (1,)), ((), ())),
            preferred_element_type=jnp.float32,
        )
        o_ref[...] = acc + b_ref[...][:, None]

    out_t = pl.pallas_call(
        mm,
        grid=(pl.cdiv(V, vt),),
        in_specs=[
            pl.BlockSpec((E, vt), lambda i: (0, i)),
            pl.BlockSpec((B, E), lambda i: (0, 0)),
            pl.BlockSpec((vt,), lambda i: (i,)),
        ],
        out_specs=pl.BlockSpec((vt, B), lambda i: (i, 0)),
        out_shape=jax.ShapeDtypeStruct((V, B), jnp.float32),
    )(lin_w.T, h, lin_b)
    return out_t.T


def kernel(x, emb_table, lin_w, lin_b):
    h = jnp.take(emb_table, x, axis=0)
    return _tc_project(h, lin_w, lin_b)


# speedup vs baseline: 2.8429x; 1.0055x over previous
import functools

import jax
import jax.numpy as jnp
from jax import lax
from jax.experimental import pallas as pl
from jax.experimental.pallas import tpu as pltpu
from jax.experimental.pallas import tpu_sc as plsc


def _sc_gather(table, idx):
    B = idx.shape[0]
    D = table.shape[1]
    info = plsc.get_sparse_core_info()
    nw = info.num_cores * info.num_subcores
    b_per_w = B // nw
    mesh = plsc.VectorSubcoreMesh(core_axis_name="c", subcore_axis_name="s")

    @functools.partial(
        pl.kernel,
        mesh=mesh,
        out_type=jax.ShapeDtypeStruct((B, D), jnp.float32),
        scratch_types=[
            pltpu.VMEM((b_per_w,), jnp.int32),
            pltpu.VMEM((b_per_w, D), jnp.float32),
            pltpu.SemaphoreType.DMA,
        ],
        compiler_params=pltpu.CompilerParams(use_tc_tiling_on_sc=False),
    )
    def gather_kernel(table_hbm, idx_hbm, out_hbm, idx_v, rows_v, sem):
        wid = lax.axis_index("s") * info.num_cores + lax.axis_index("c")
        base = wid * b_per_w
        pltpu.sync_copy(idx_hbm.at[pl.ds(base, b_per_w)], idx_v)
        pltpu.async_copy(table_hbm.at[idx_v], rows_v, sem).wait()
        pltpu.sync_copy(rows_v, out_hbm.at[pl.ds(base, b_per_w)])

    return gather_kernel(table, idx)


def _tc_project(h, lin_w, lin_b, vt=4096):
    B, E = h.shape
    V = lin_w.shape[0]

    def mm(w_ref, h_ref, b_ref, o_ref):
        acc = lax.dot_general(
            w_ref[...], h_ref[...],
            (((0,), (1,)), ((), ())),
            preferred_element_type=jnp.float32,
        )
        o_ref[...] = acc + b_ref[...][:, None]

    out_t = pl.pallas_call(
        mm,
        grid=(pl.cdiv(V, vt),),
        in_specs=[
            pl.BlockSpec((E, vt), lambda i: (0, i)),
            pl.BlockSpec((B, E), lambda i: (0, 0)),
            pl.BlockSpec((vt,), lambda i: (i,)),
        ],
        out_specs=pl.BlockSpec((vt, B), lambda i: (i, 0)),
        out_shape=jax.ShapeDtypeStruct((V, B), jnp.float32),
    )(lin_w.T, h, lin_b)
    return out_t.T


def kernel(x, emb_table, lin_w, lin_b):
    h = jnp.take(emb_table, x, axis=0)
    return _tc_project(h, lin_w, lin_b)
